# Initial kernel scaffold; baseline (speedup 1.0000x reference)
import jax
import jax.numpy as jnp
from jax.experimental import pallas as pl


def kernel(z, targets, centers):
    raise NotImplementedError("write your pallas kernel here")


# speedup vs baseline: 3.8366x; 3.8366x over previous
import functools
import jax
import jax.numpy as jnp
from jax.experimental import pallas as pl
from jax.experimental.pallas import tpu as pltpu

_NUM_CLASSES = 5
_LAMBDA_C = 0.01
_BLOCK = 2048


def _body(z_ref, t_ref, c_ref, out_ref):
    i = pl.program_id(0)
    t = t_ref[0]
    z = z_ref[...]
    tcol = t.reshape(_BLOCK, 1)
    bc = jnp.zeros_like(z)
    for k in range(_NUM_CLASSES):
        mask = (tcol == k).astype(jnp.float32)
        bc = bc + mask * c_ref[k, :][None, :]
    d = z - bc
    partial = jnp.sum(d * d)

    @pl.when(i == 0)
    def _init():
        out_ref[0, 0] = 0.0

    out_ref[0, 0] += partial


def kernel(z, targets, centers):
    batch = z.shape[0]
    nblocks = batch // _BLOCK
    t3 = targets.astype(jnp.int32).reshape(nblocks, 1, _BLOCK)
    total = pl.pallas_call(
        _body,
        grid=(nblocks,),
        in_specs=[
            pl.BlockSpec((_BLOCK, z.shape[1]), lambda i: (i, 0)),
            pl.BlockSpec((1, 1, _BLOCK), lambda i: (i, 0, 0)),
            pl.BlockSpec(centers.shape, lambda i: (0, 0)),
        ],
        out_specs=pl.BlockSpec(memory_space=pltpu.SMEM),
        out_shape=jax.ShapeDtypeStruct((1, 1), jnp.float32),
    )(z, t3, centers)
    return _LAMBDA_C * 0.5 * total[0, 0] / batch
